# baseline (speedup 1.0000x reference)
import functools

import jax
import jax.numpy as jnp
from jax import lax
from jax.experimental import pallas as pl
from jax.experimental.pallas import tpu as pltpu
from jax.experimental.pallas import tpu_sc as plsc

D = 64
HID = 128
CLS = 2
B = 4096
HIST = 200

NC, NS = 2, 16
NW = NC * NS
ROWS = B // NW
GCHUNK = HIST // 2
NG = ROWS * 2
HALF = ROWS // 2
L = 16
INV = 1.0 / HIST


def _make_pool_kernel():
    mesh = plsc.VectorSubcoreMesh(core_axis_name="c", subcore_axis_name="s")

    @functools.partial(
        pl.kernel,
        mesh=mesh,
        out_type=jax.ShapeDtypeStruct((NW, ROWS, D), jnp.float32),
        scratch_types=[
            pltpu.VMEM((NG, GCHUNK), jnp.int32),
            pltpu.VMEM((GCHUNK, D), jnp.float32),
            pltpu.VMEM((GCHUNK, D), jnp.float32),
            pltpu.VMEM((GCHUNK, D), jnp.float32),
            pltpu.VMEM((GCHUNK, D), jnp.float32),
            pltpu.VMEM((ROWS, D), jnp.float32),
            pltpu.SemaphoreType.DMA,
            pltpu.SemaphoreType.DMA,
            pltpu.SemaphoreType.DMA,
            pltpu.SemaphoreType.DMA,
        ],
        compiler_params=pltpu.CompilerParams(use_tc_tiling_on_sc=False),
    )
    def pool(x_hbm, emb_hbm, out_hbm, idx_v, b0, b1, b2, b3, acc_v,
             s0, s1, s2, s3):
        wid = lax.axis_index("s") * NC + lax.axis_index("c")
        pltpu.sync_copy(x_hbm.at[wid], idx_v)

        def fire(g, buf, sem):
            pltpu.make_async_copy(emb_hbm.at[idx_v.at[g]], buf, sem).start()

        def wait(buf, sem):
            pltpu.make_async_copy(emb_hbm.at[idx_v.at[0]], buf, sem).wait()

        def accum(buf, acc):
            def body(j, a):
                return (a[0] + buf[j, pl.ds(0, L)],
                        a[1] + buf[j, pl.ds(L, L)],
                        a[2] + buf[j, pl.ds(2 * L, L)],
                        a[3] + buf[j, pl.ds(3 * L, L)])
            return lax.fori_loop(0, GCHUNK, body, acc)

        def store_row(r, a):
            acc_v[r, pl.ds(0, L)] = a[0] * INV
            acc_v[r, pl.ds(L, L)] = a[1] * INV
            acc_v[r, pl.ds(2 * L, L)] = a[2] * INV
            acc_v[r, pl.ds(3 * L, L)] = a[3] * INV

        zeros = (jnp.zeros((L,), jnp.float32),) * 4

        fire(0, b0, s0)
        fire(1, b1, s1)
        fire(2, b2, s2)
        fire(3, b3, s3)

        def pair_body(bb, carry):
            g = 4 * bb
            not_last = bb < HALF - 1

            wait(b0, s0)
            acc = accum(b0, zeros)

            @pl.when(not_last)
            def _():
                fire(g + 4, b0, s0)

            wait(b1, s1)
            acc = accum(b1, acc)

            @pl.when(not_last)
            def _():
                fire(g + 5, b1, s1)

            store_row(2 * bb, acc)

            wait(b2, s2)
            acc2 = accum(b2, zeros)

            @pl.when(not_last)
            def _():
                fire(g + 6, b2, s2)

            wait(b3, s3)
            acc2 = accum(b3, acc2)

            @pl.when(not_last)
            def _():
                fire(g + 7, b3, s3)

            store_row(2 * bb + 1, acc2)
            return carry

        lax.fori_loop(0, HALF, pair_body, 0)
        pltpu.sync_copy(acc_v, out_hbm.at[wid])

    return pool


_pool = _make_pool_kernel()


def _mlp_body(x_ref, w1t_ref, b1_ref, w2t_ref, b2_ref, o_ref):
    h = jnp.dot(x_ref[...], w1t_ref[...], preferred_element_type=jnp.float32)
    h = jnp.maximum(h + b1_ref[...], 0.0)
    o_ref[...] = (jnp.dot(h, w2t_ref[...], preferred_element_type=jnp.float32)
                  + b2_ref[...])


def kernel(x_in, emb, W1, b1, W2, b2):
    x3 = x_in.reshape(NW, NG, GCHUNK)
    pooled = _pool(x3, emb).reshape(B, D)
    logits = pl.pallas_call(
        _mlp_body,
        out_shape=jax.ShapeDtypeStruct((B, CLS), jnp.float32),
    )(pooled, W1.T, b1.reshape(1, HID), W2.T, b2.reshape(1, CLS))
    return logits
